# Initial kernel scaffold; baseline (speedup 1.0000x reference)
import jax
import jax.numpy as jnp
from jax.experimental import pallas as pl


def kernel(z, nodes):
    raise NotImplementedError("write your pallas kernel here")


# speedup vs baseline: 2.5293x; 2.5293x over previous
import functools

import jax
import jax.numpy as jnp
from jax import lax
from jax.experimental import pallas as pl
from jax.experimental.pallas import tpu as pltpu
from jax.experimental.pallas import tpu_sc as plsc

GRID_H = 32
GRID_W = 32
LATENT_DIM = 64
ALPHA = 1.0
TIME_DECAY = 0.9
MAX_SEQ_LEN = 4000

_ROW_BLOCK = 256


def _dist_body(z_ref, w_ref, nodes_ref, q_ref, bmu_ref):
    z = z_ref[...]
    wz = z * w_ref[...]
    nodes = nodes_ref[...]
    x2 = jnp.sum(wz * wz, axis=1, keepdims=True)
    n2 = jnp.sum(nodes * nodes, axis=1)[None, :]
    g = lax.dot_general(
        wz, nodes, (((1,), (1,)), ((), ())),
        precision=lax.Precision.HIGHEST,
        preferred_element_type=jnp.float32,
    )
    d2 = (x2 + n2) - 2.0 * g
    dist = jnp.sqrt(jnp.maximum(d2, 0.0))
    q = 1.0 / (1.0 + dist / ALPHA)
    q = q / jnp.maximum(jnp.sum(q, axis=1, keepdims=True), 1e-12)
    q_ref[...] = q

    idx = lax.broadcasted_iota(jnp.int32, d2.shape, 1)
    md = jnp.min(d2, axis=1, keepdims=True)
    bmu = jnp.min(jnp.where(d2 == md, idx, jnp.int32(2 ** 30)), axis=1)
    bmu_ref[0, 0, :] = bmu


def _dist_call(z_flat, w_bcast, nodes):
    n, d = z_flat.shape
    k = nodes.shape[0]
    r = _ROW_BLOCK
    nblk = n // r
    return pl.pallas_call(
        _dist_body,
        grid=(nblk,),
        in_specs=[
            pl.BlockSpec((r, d), lambda i: (i, 0)),
            pl.BlockSpec((r, d), lambda i: (i, 0)),
            pl.BlockSpec((k, d), lambda i: (0, 0)),
        ],
        out_specs=[
            pl.BlockSpec((r, k), lambda i: (i, 0)),
            pl.BlockSpec((1, 1, r), lambda i: (i, 0, 0)),
        ],
        out_shape=[
            jax.ShapeDtypeStruct((n, k), jnp.float32),
            jax.ShapeDtypeStruct((nblk, 1, r), jnp.int32),
        ],
    )(z_flat, w_bcast, nodes)


def _som_update_sc(nodes, bmu_flat, z_flat):
    n, d = z_flat.shape
    info = plsc.get_sparse_core_info()
    nc, ns = info.num_cores, info.num_subcores
    nw = nc * ns
    bpw = n // nw
    mesh = plsc.VectorSubcoreMesh(core_axis_name="c", subcore_axis_name="s")

    @functools.partial(
        pl.kernel,
        mesh=mesh,
        out_type=jax.ShapeDtypeStruct((n, d), jnp.float32),
        compiler_params=pltpu.CompilerParams(use_tc_tiling_on_sc=False),
        scratch_types=[
            pltpu.VMEM((bpw,), jnp.int32),
            pltpu.VMEM((bpw, d), jnp.float32),
            pltpu.VMEM((bpw, d), jnp.float32),
            pltpu.SemaphoreType.DMA,
        ],
    )
    def sc_body(nodes_hbm, idx_hbm, z_hbm, out_hbm, idx_v, g_v, z_v, sem):
        wid = lax.axis_index("s") * nc + lax.axis_index("c")
        base = wid * bpw
        pltpu.sync_copy(idx_hbm.at[pl.ds(base, bpw)], idx_v)
        gather = pltpu.async_copy(nodes_hbm.at[idx_v], g_v, sem)
        pltpu.sync_copy(z_hbm.at[pl.ds(base, bpw)], z_v)
        gather.wait()

        def row_body(r, carry):
            for c in range(d // 16):
                sl = pl.ds(c * 16, 16)
                zv = z_v[r, sl]
                gv = g_v[r, sl]
                z_v[r, sl] = zv + 0.1 * (gv - zv)
            return carry

        lax.fori_loop(0, bpw, row_body, 0)
        pltpu.sync_copy(z_v, out_hbm.at[pl.ds(base, bpw)])

    return sc_body(nodes, bmu_flat, z_flat)


def kernel(z, nodes):
    b, t, d = z.shape
    n = b * t
    ts = jnp.arange(MAX_SEQ_LEN, dtype=jnp.float32)
    decay = jnp.power(jnp.float32(TIME_DECAY), jnp.float32(MAX_SEQ_LEN) - ts - 1.0)
    w = decay[MAX_SEQ_LEN - t:]
    w_rows = jnp.tile(w, (b,))[:, None]
    w_bcast = jnp.broadcast_to(w_rows, (n, d))

    z_flat = z.reshape(n, d)
    q, bmu_blocks = _dist_call(z_flat, w_bcast, nodes)
    bmu_flat = bmu_blocks.reshape(n)
    som_flat = _som_update_sc(nodes, bmu_flat, z_flat)
    bmu_indices = bmu_flat.reshape(b, t)
    som_z = som_flat.reshape(b, t, d)
    nodes_grid = nodes.reshape(GRID_H, GRID_W, -1)
    return som_z, q, bmu_indices, nodes_grid
